# baseline (speedup 1.0000x reference)
import dataclasses
import functools

import jax
import jax.numpy as jnp
from jax import lax
from jax.experimental import pallas as pl
from jax.experimental.pallas import tpu as pltpu
from jax.experimental.pallas import tpu_sc as plsc

N = 10000
D = 128
E = 320000
C_OUT = 64
NC = 2
NS = 16
NW = NC * NS
EPW = E // NW
CH = 80
NCHUNK = EPW // CH
RPT = 624
TAIL = N - NS * RPT
CL = 16

_mesh = plsc.VectorSubcoreMesh(core_axis_name="c", subcore_axis_name="s")


def _copy_rows(s, src_at, dst_at):
    rbase = s * RPT
    pltpu.sync_copy(src_at(pl.ds(rbase, RPT)), dst_at(pl.ds(rbase, RPT)))

    @pl.when(s == 0)
    def _():
        pltpu.sync_copy(src_at(pl.ds(NS * RPT, TAIL)),
                        dst_at(pl.ds(NS * RPT, TAIL)))


NBUF = 3
NTRASH = N


def _make_sc_agg(ch, with_count):
    nchunk = -(-EPW // ch)
    ept = nchunk * ch
    nmain = (nchunk // NBUF) * NBUF

    def body(table, src_h, dst_h, zfeat_h, *rest):
        if with_count:
            (zcnt_h, acc_out, cnt_out, cnt_local, acc_sh, *rings) = rest
        else:
            (acc_out, acc_sh, *rings) = rest
        srcv = rings[0:NBUF]
        dstv = rings[NBUF:2 * NBUF]
        dsc = rings[2 * NBUF:3 * NBUF]
        isems = rings[3 * NBUF:4 * NBUF]
        rows = rings[4 * NBUF:5 * NBUF]
        gsems = rings[5 * NBUF:6 * NBUF]
        ssems = rings[6 * NBUF:7 * NBUF]

        c = lax.axis_index("c")
        s = lax.axis_index("s")
        wid = s * NC + c
        ebase = wid * ept
        _copy_rows(s, lambda d: zfeat_h.at[d], lambda d: acc_sh.at[d])
        if with_count:
            pltpu.sync_copy(zcnt_h, cnt_local.at[pl.ds(0, N)])
        plsc.subcore_barrier()

        ones = jnp.full((16,), 1.0, jnp.float32)

        def count_fn(dref):
            if with_count:
                for g in range(ch // 16):
                    plsc.addupdate_scatter(
                        cnt_local, [dref[pl.ds(g * 16, 16)]], ones)

        def start_idx(k, b):
            o = ebase + k * ch
            pltpu.make_async_copy(src_h.at[pl.ds(o, ch)], srcv[b],
                                  isems[b]).start()
            pltpu.make_async_copy(dst_h.at[pl.ds(o, ch)], dstv[b],
                                  isems[b]).start()

        def wait_idx(b):
            pltpu.make_async_copy(src_h.at[pl.ds(0, ch)], srcv[b],
                                  isems[b]).wait()
            pltpu.make_async_copy(dst_h.at[pl.ds(0, ch)], dstv[b],
                                  isems[b]).wait()

        def start_gather(b):
            wait_idx(b)
            pltpu.make_async_copy(table.at[srcv[b]], rows[b],
                                  gsems[b]).start()

        def wait_scatter(b):
            pltpu.make_async_copy(rows[b], acc_sh.at[dsc[b]],
                                  ssems[b]).wait()

        def drain(k, b):
            pltpu.make_async_copy(table.at[pl.ds(0, ch)], rows[b],
                                  gsems[b]).wait()
            for g in range(ch // 16):
                dsc[b][pl.ds(g * 16, 16)] = dstv[b][pl.ds(g * 16, 16)]
            pltpu.async_copy(rows[b], acc_sh.at[dsc[b]], ssems[b], add=True)
            count_fn(dsc[b])

        for b in range(NBUF):
            start_idx(b, b)
        for b in range(NBUF - 1):
            start_gather(b)

        @pl.loop(0, nmain, step=NBUF)
        def _(ci):
            for b in range(NBUF):
                k = ci + b
                drain(k, b)
                nk = k + NBUF

                @pl.when(nk < nchunk)
                def _():
                    start_idx(nk, b)

                m = k + NBUF - 1
                mb = (b + NBUF - 1) % NBUF

                @pl.when(m < nchunk)
                def _():
                    @pl.when(k > 0)
                    def _():
                        wait_scatter(mb)

                    start_gather(mb)

        for t in range(nmain, nchunk):
            drain(t, t % NBUF)
        for b in range(NBUF):
            wait_scatter(b)

        plsc.subcore_barrier()
        _copy_rows(s, lambda d: acc_sh.at[d], lambda d: acc_out.at[c, d])
        if with_count:
            pltpu.sync_copy(cnt_local.at[pl.ds(0, N)],
                            cnt_out.at[pl.ds(wid * N, N)])

    out_type = jax.ShapeDtypeStruct((NC, N, D), jnp.float32)
    if with_count:
        out_type = (out_type, jax.ShapeDtypeStruct((NW * N,), jnp.float32))
    scratch = (
        ([pltpu.VMEM((N + 16,), jnp.float32)] if with_count else [])
        + [pltpu.VMEM_SHARED((N + 16, D), jnp.float32)]
        + [pltpu.VMEM((ch,), jnp.int32)] * (3 * NBUF)
        + [pltpu.SemaphoreType.DMA] * NBUF
        + [pltpu.VMEM((ch, D), jnp.float32)] * NBUF
        + [pltpu.SemaphoreType.DMA] * (2 * NBUF)
    )
    cp = pltpu.CompilerParams()
    if "needs_layout_passes" in pltpu.CompilerParams.__dataclass_fields__:
        cp = dataclasses.replace(cp, needs_layout_passes=False)
    return pl.kernel(body, out_type=out_type, mesh=_mesh,
                     scratch_types=scratch, compiler_params=cp), ept


_sc_agg_count, _EPT1 = _make_sc_agg(96, with_count=True)
_sc_agg, _EPT2 = _make_sc_agg(128, with_count=False)


def _pad_edges(src, dst, ept):
    if ept == EPW:
        return src, dst
    pad = ept - EPW
    src_p = jnp.concatenate(
        [src.reshape(NW, EPW), jnp.zeros((NW, pad), jnp.int32)], axis=1)
    dst_p = jnp.concatenate(
        [dst.reshape(NW, EPW), jnp.full((NW, pad), NTRASH, jnp.int32)], axis=1)
    return src_p.reshape(-1), dst_p.reshape(-1)


_DN = (((1,), (1,)), ((), ()))


def _dense1_kernel(acc, cnt, x, w1l, b1l, w1r, out):
    a = acc[...]
    ssum = a[0] + a[1]
    deg = jnp.sum(cnt[...].reshape(NW, -1), axis=0)[:, None]
    mean = ssum / jnp.maximum(deg, 1.0)
    h = lax.dot_general(mean, w1l[...], _DN, preferred_element_type=jnp.float32)
    h = h + lax.dot_general(x[...], w1r[...], _DN,
                            preferred_element_type=jnp.float32)
    h = h + b1l[...]
    out[...] = jnp.maximum(h, 0.0)


def _dense2_kernel(acc, cnt, h1, w2l, b2l, w2r, wlin, blin, out):
    a = acc[...]
    ssum = a[0] + a[1]
    deg = jnp.sum(cnt[...].reshape(NW, -1), axis=0)[:, None]
    mean = ssum / jnp.maximum(deg, 1.0)
    h = lax.dot_general(mean, w2l[...], _DN, preferred_element_type=jnp.float32)
    h = h + lax.dot_general(h1[...], w2r[...], _DN,
                            preferred_element_type=jnp.float32)
    h = jnp.maximum(h + b2l[...], 0.0)
    out[...] = lax.dot_general(h, wlin[...], _DN,
                               preferred_element_type=jnp.float32) + blin[...]


_R = 1000

_dense1 = pl.pallas_call(
    _dense1_kernel,
    grid=(N // _R,),
    in_specs=[
        pl.BlockSpec((NC, _R, D), lambda i: (0, i, 0)),
        pl.BlockSpec((NW, 1, 1, _R), lambda i: (0, i, 0, 0)),
        pl.BlockSpec((_R, D), lambda i: (i, 0)),
        pl.BlockSpec((D, D), lambda i: (0, 0)),
        pl.BlockSpec((1, D), lambda i: (0, 0)),
        pl.BlockSpec((D, D), lambda i: (0, 0)),
    ],
    out_specs=pl.BlockSpec((_R, D), lambda i: (i, 0)),
    out_shape=jax.ShapeDtypeStruct((N, D), jnp.float32),
)

_dense2 = pl.pallas_call(
    _dense2_kernel,
    grid=(N // _R,),
    in_specs=[
        pl.BlockSpec((NC, _R, D), lambda i: (0, i, 0)),
        pl.BlockSpec((NW, 1, 1, _R), lambda i: (0, i, 0, 0)),
        pl.BlockSpec((_R, D), lambda i: (i, 0)),
        pl.BlockSpec((D, D), lambda i: (0, 0)),
        pl.BlockSpec((1, D), lambda i: (0, 0)),
        pl.BlockSpec((D, D), lambda i: (0, 0)),
        pl.BlockSpec((C_OUT, D), lambda i: (0, 0)),
        pl.BlockSpec((1, C_OUT), lambda i: (0, 0)),
    ],
    out_specs=pl.BlockSpec((_R, C_OUT), lambda i: (i, 0)),
    out_shape=jax.ShapeDtypeStruct((N, C_OUT), jnp.float32),
)


def kernel(x, edge_index, W1l, b1l, W1r, W2l, b2l, W2r, Wlin, blin):
    src = edge_index[0]
    dst = edge_index[1]
    src1, dst1 = _pad_edges(src, dst, _EPT1)
    src2, dst2 = _pad_edges(src, dst, _EPT2)
    zfeat = jnp.zeros((N, D), jnp.float32)
    zcnt = jnp.zeros((N,), jnp.float32)
    acc1, cnt = _sc_agg_count(x, src1, dst1, zfeat, zcnt)
    cnt = cnt.reshape(NW, N // _R, 1, _R)
    h1 = _dense1(acc1, cnt, x, W1l, b1l.reshape(1, D), W1r)
    acc2 = _sc_agg(h1, src2, dst2, zfeat)
    return _dense2(acc2, cnt, h1, W2l, b2l.reshape(1, D), W2r,
                   Wlin, blin.reshape(1, C_OUT))


# speedup vs baseline: 1.8513x; 1.8513x over previous
import dataclasses
import functools

import jax
import jax.numpy as jnp
from jax import lax
from jax.experimental import pallas as pl
from jax.experimental.pallas import tpu as pltpu
from jax.experimental.pallas import tpu_sc as plsc

N = 10000
D = 128
E = 320000
C_OUT = 64
NC = 2
NS = 16
NW = NC * NS
EPW = E // NW
CH = 80
NCHUNK = EPW // CH
RPT = 624
TAIL = N - NS * RPT
CL = 16

_mesh = plsc.VectorSubcoreMesh(core_axis_name="c", subcore_axis_name="s")


def _copy_rows(s, src_at, dst_at):
    rbase = s * RPT
    pltpu.sync_copy(src_at(pl.ds(rbase, RPT)), dst_at(pl.ds(rbase, RPT)))

    @pl.when(s == 0)
    def _():
        pltpu.sync_copy(src_at(pl.ds(NS * RPT, TAIL)),
                        dst_at(pl.ds(NS * RPT, TAIL)))


NTRASH = N


def _make_sc_agg(ch, with_count, NBUF):
    nchunk = -(-EPW // ch)
    ept = nchunk * ch
    nmain = (nchunk // NBUF) * NBUF

    def body(table, src_h, dst_h, zfeat_h, *rest):
        if with_count:
            (zcnt_h, acc_out, cnt_out, cnt_local, acc_sh, *rings) = rest
        else:
            (acc_out, acc_sh, *rings) = rest
        srcv = rings[0:NBUF]
        dstv = rings[NBUF:2 * NBUF]
        dsc = rings[2 * NBUF:3 * NBUF]
        isems = rings[3 * NBUF:4 * NBUF]
        rows = rings[4 * NBUF:5 * NBUF]
        gsems = rings[5 * NBUF:6 * NBUF]
        ssems = rings[6 * NBUF:7 * NBUF]

        c = lax.axis_index("c")
        s = lax.axis_index("s")
        wid = s * NC + c
        ebase = wid * ept
        _copy_rows(s, lambda d: zfeat_h.at[d], lambda d: acc_sh.at[d])
        if with_count:
            pltpu.sync_copy(zcnt_h, cnt_local.at[pl.ds(0, N)])
        plsc.subcore_barrier()

        ones = jnp.full((16,), 1.0, jnp.float32)

        def count_fn(dref):
            if with_count:
                for g in range(ch // 16):
                    plsc.addupdate_scatter(
                        cnt_local, [dref[pl.ds(g * 16, 16)]], ones)

        def start_idx(k, b):
            o = ebase + k * ch
            pltpu.make_async_copy(src_h.at[pl.ds(o, ch)], srcv[b],
                                  isems[b]).start()
            pltpu.make_async_copy(dst_h.at[pl.ds(o, ch)], dstv[b],
                                  isems[b]).start()

        def wait_idx(b):
            pltpu.make_async_copy(src_h.at[pl.ds(0, ch)], srcv[b],
                                  isems[b]).wait()
            pltpu.make_async_copy(dst_h.at[pl.ds(0, ch)], dstv[b],
                                  isems[b]).wait()

        def start_gather(b):
            wait_idx(b)
            pltpu.make_async_copy(table.at[srcv[b]], rows[b],
                                  gsems[b]).start()

        def wait_scatter(b):
            pltpu.make_async_copy(rows[b], acc_sh.at[dsc[b]],
                                  ssems[b]).wait()

        def drain(k, b):
            pltpu.make_async_copy(table.at[pl.ds(0, ch)], rows[b],
                                  gsems[b]).wait()
            for g in range(ch // 16):
                dsc[b][pl.ds(g * 16, 16)] = dstv[b][pl.ds(g * 16, 16)]
            pltpu.async_copy(rows[b], acc_sh.at[dsc[b]], ssems[b], add=True)
            count_fn(dsc[b])

        for b in range(NBUF):
            start_idx(b, b)
        for b in range(NBUF - 1):
            start_gather(b)

        @pl.loop(0, nmain, step=NBUF)
        def _(ci):
            for b in range(NBUF):
                k = ci + b
                drain(k, b)
                nk = k + NBUF

                @pl.when(nk < nchunk)
                def _():
                    start_idx(nk, b)

                m = k + NBUF - 1
                mb = (b + NBUF - 1) % NBUF

                @pl.when(m < nchunk)
                def _():
                    @pl.when(k > 0)
                    def _():
                        wait_scatter(mb)

                    start_gather(mb)

        for t in range(nmain, nchunk):
            drain(t, t % NBUF)
        for b in range(NBUF):
            wait_scatter(b)

        plsc.subcore_barrier()
        _copy_rows(s, lambda d: acc_sh.at[d], lambda d: acc_out.at[c, d])
        if with_count:
            pltpu.sync_copy(cnt_local.at[pl.ds(0, N)],
                            cnt_out.at[pl.ds(wid * N, N)])

    out_type = jax.ShapeDtypeStruct((NC, N, D), jnp.float32)
    if with_count:
        out_type = (out_type, jax.ShapeDtypeStruct((NW * N,), jnp.float32))
    scratch = (
        ([pltpu.VMEM((N + 16,), jnp.float32)] if with_count else [])
        + [pltpu.VMEM_SHARED((N + 16, D), jnp.float32)]
        + [pltpu.VMEM((ch,), jnp.int32)] * (3 * NBUF)
        + [pltpu.SemaphoreType.DMA] * NBUF
        + [pltpu.VMEM((ch, D), jnp.float32)] * NBUF
        + [pltpu.SemaphoreType.DMA] * (2 * NBUF)
    )
    cp = pltpu.CompilerParams()
    if "needs_layout_passes" in pltpu.CompilerParams.__dataclass_fields__:
        cp = dataclasses.replace(cp, needs_layout_passes=False)
    return pl.kernel(body, out_type=out_type, mesh=_mesh,
                     scratch_types=scratch, compiler_params=cp), ept


_sc_agg_count, _EPT1 = _make_sc_agg(80, with_count=True, NBUF=3)
_sc_agg, _EPT2 = _make_sc_agg(80, with_count=False, NBUF=4)


def _pad_edges(src, dst, ept):
    if ept == EPW:
        return src, dst
    pad = ept - EPW
    src_p = jnp.concatenate(
        [src.reshape(NW, EPW), jnp.zeros((NW, pad), jnp.int32)], axis=1)
    dst_p = jnp.concatenate(
        [dst.reshape(NW, EPW), jnp.full((NW, pad), NTRASH, jnp.int32)], axis=1)
    return src_p.reshape(-1), dst_p.reshape(-1)


_DN = (((1,), (1,)), ((), ()))


def _self_kernel(x, w, b, out):
    out[...] = lax.dot_general(x[...], w[...], _DN,
                               preferred_element_type=jnp.float32) + b[...]


def _mean(acc, cnt):
    a = acc[...]
    deg = jnp.sum(cnt[...].reshape(NW, -1), axis=0)[:, None]
    return (a[0] + a[1]) / jnp.maximum(deg, 1.0)


def _dense1_kernel(acc, cnt, pre, w1l, out):
    h = lax.dot_general(_mean(acc, cnt), w1l[...], _DN,
                        preferred_element_type=jnp.float32)
    out[...] = jnp.maximum(h + pre[...], 0.0)


def _dense2_kernel(acc, cnt, pre, w2l, wlin, blin, out):
    h = lax.dot_general(_mean(acc, cnt), w2l[...], _DN,
                        preferred_element_type=jnp.float32)
    h = jnp.maximum(h + pre[...], 0.0)
    out[...] = lax.dot_general(h, wlin[...], _DN,
                               preferred_element_type=jnp.float32) + blin[...]


_R = 1000

_ACC_SPEC = pl.BlockSpec((NC, _R, D), lambda i: (0, i, 0))
_CNT_SPEC = pl.BlockSpec((NW, 1, 1, _R), lambda i: (0, i, 0, 0))
_ROW_SPEC = pl.BlockSpec((_R, D), lambda i: (i, 0))
_W_SPEC = pl.BlockSpec((D, D), lambda i: (0, 0))
_B_SPEC = pl.BlockSpec((1, D), lambda i: (0, 0))
_ROWS_F32 = jax.ShapeDtypeStruct((N, D), jnp.float32)

_self = pl.pallas_call(
    _self_kernel,
    grid=(N // _R,),
    in_specs=[_ROW_SPEC, _W_SPEC, _B_SPEC],
    out_specs=_ROW_SPEC,
    out_shape=_ROWS_F32,
)

_dense1 = pl.pallas_call(
    _dense1_kernel,
    grid=(N // _R,),
    in_specs=[_ACC_SPEC, _CNT_SPEC, _ROW_SPEC, _W_SPEC],
    out_specs=_ROW_SPEC,
    out_shape=_ROWS_F32,
)

_dense2 = pl.pallas_call(
    _dense2_kernel,
    grid=(N // _R,),
    in_specs=[_ACC_SPEC, _CNT_SPEC, _ROW_SPEC, _W_SPEC,
              pl.BlockSpec((C_OUT, D), lambda i: (0, 0)),
              pl.BlockSpec((1, C_OUT), lambda i: (0, 0))],
    out_specs=pl.BlockSpec((_R, C_OUT), lambda i: (i, 0)),
    out_shape=jax.ShapeDtypeStruct((N, C_OUT), jnp.float32),
)


def kernel(x, edge_index, W1l, b1l, W1r, W2l, b2l, W2r, Wlin, blin):
    src = edge_index[0]
    dst = edge_index[1]
    src1, dst1 = _pad_edges(src, dst, _EPT1)
    src2, dst2 = _pad_edges(src, dst, _EPT2)
    zfeat = jnp.zeros((N, D), jnp.float32)
    zcnt = jnp.zeros((N,), jnp.float32)
    acc1, cnt = _sc_agg_count(x, src1, dst1, zfeat, zcnt)
    pre1 = _self(x, W1r, b1l.reshape(1, D))
    cnt = cnt.reshape(NW, N // _R, 1, _R)
    h1 = _dense1(acc1, cnt, pre1, W1l)
    acc2 = _sc_agg(h1, src2, dst2, zfeat)
    pre2 = _self(h1, W2r, b2l.reshape(1, D))
    return _dense2(acc2, cnt, pre2, W2l, Wlin, blin.reshape(1, C_OUT))
